# Initial kernel scaffold; baseline (speedup 1.0000x reference)
import jax
import jax.numpy as jnp
from jax.experimental import pallas as pl


def kernel(node_feat, src, dst, neg, W1, b1, W2, b2, W3, b3, g1, be1, g2, be2, Ws, bs, Wd, bd, Wf, bf):
    raise NotImplementedError("write your pallas kernel here")


# speedup vs baseline: 3.2405x; 3.2405x over previous
import dataclasses
import functools

import jax
import jax.numpy as jnp
import numpy as np
from jax import lax
from jax.experimental import pallas as pl
from jax.experimental.pallas import tpu as pltpu
from jax.experimental.pallas import tpu_sc as plsc

N = 10000
E = 320000
D = 128
NP_ = 10240
NC = 2
NS = 16
NW = NC * NS
EPT = E // NW
CH = 128
NCH = 80
SB = 16
NSB = NCH // SB
EPT2 = NCH * CH
E2 = NW * EPT2
ROWS_ACC = NP_ // NS
ZR = 128
PAD_DST = NP_ - 1

_mesh = plsc.VectorSubcoreMesh(core_axis_name="c", subcore_axis_name="s")

_cp = pltpu.CompilerParams()
if "needs_layout_passes" in pltpu.CompilerParams.__dataclass_fields__:
    _cp = dataclasses.replace(_cp, needs_layout_passes=False)

BT = 1024
_BN_SCALE = float(1.0 / np.sqrt(1.0 + 1e-5))


@functools.partial(
    pl.kernel,
    out_type=jax.ShapeDtypeStruct((NC, NP_, D), jnp.float32),
    mesh=_mesh,
    scratch_types=[
        pltpu.VMEM((SB, CH), jnp.int32),
        pltpu.VMEM((SB, CH), jnp.int32),
        pltpu.VMEM((CH, D), jnp.float32),
        pltpu.VMEM((CH, D), jnp.float32),
        pltpu.VMEM_SHARED((NP_, D), jnp.float32),
        pltpu.SemaphoreType.DMA,
        pltpu.SemaphoreType.DMA,
    ],
)
def _agg_kernel(zeros_hbm, src_hbm, dst_hbm, y_hbm, out_hbm,
                sidx, didx, buf0, buf1, acc, sem0, sem1):
    cid = lax.axis_index("c")
    sid = lax.axis_index("s")
    wid = cid * NS + sid
    for z in range(ROWS_ACC // ZR):
        pltpu.sync_copy(zeros_hbm, acc.at[pl.ds(sid * ROWS_ACC + z * ZR, ZR)])
    plsc.subcore_barrier()

    @pl.loop(0, NSB)
    def _(sb):
        pltpu.sync_copy(src_hbm.at[wid, pl.ds(sb * SB, SB)], sidx)
        pltpu.sync_copy(dst_hbm.at[wid, pl.ds(sb * SB, SB)], didx)
        pltpu.async_copy(y_hbm.at[sidx.at[0]], buf0, sem0)
        for k in range(0, SB, 2):
            pltpu.make_async_copy(y_hbm.at[sidx.at[k]], buf0, sem0).wait()
            pltpu.async_copy(y_hbm.at[sidx.at[k + 1]], buf1, sem1)
            pltpu.sync_copy(buf0, acc.at[didx.at[k]], add=True)
            pltpu.make_async_copy(y_hbm.at[sidx.at[k + 1]], buf1, sem1).wait()
            if k + 2 < SB:
                pltpu.async_copy(y_hbm.at[sidx.at[k + 2]], buf0, sem0)
            pltpu.sync_copy(buf1, acc.at[didx.at[k + 1]], add=True)

    plsc.subcore_barrier()
    pltpu.sync_copy(acc.at[pl.ds(sid * ROWS_ACC, ROWS_ACC)],
                    out_hbm.at[cid, pl.ds(sid * ROWS_ACC, ROWS_ACC)])


@functools.partial(
    pl.kernel,
    out_type=(jax.ShapeDtypeStruct((E2,), jnp.float32),
              jax.ShapeDtypeStruct((E2,), jnp.float32)),
    mesh=_mesh,
    scratch_types=[
        pltpu.VMEM((SB, CH), jnp.int32),
        pltpu.VMEM((SB, CH), jnp.int32),
        pltpu.VMEM((SB, CH), jnp.int32),
        pltpu.VMEM((CH, D), jnp.float32),
        pltpu.VMEM((CH, D), jnp.float32),
        pltpu.VMEM((CH, D), jnp.float32),
        pltpu.VMEM((D,), jnp.float32),
        pltpu.VMEM((16,), jnp.float32),
        pltpu.VMEM((EPT2,), jnp.float32),
        pltpu.VMEM((EPT2,), jnp.float32),
        pltpu.SemaphoreType.DMA,
        pltpu.SemaphoreType.DMA,
        pltpu.SemaphoreType.DMA,
    ],
    compiler_params=_cp,
)
def _decode_kernel(s_hbm, d_hbm, src_hbm, dst_hbm, neg_hbm, wf_hbm, bf_hbm,
                   pos_hbm, negout_hbm,
                   sidx, didx, nidx, sbuf, dpbuf, dnbuf, wf_v, bf_v,
                   pos_v, neg_v, sem0, sem1, sem2):
    cid = lax.axis_index("c")
    sid = lax.axis_index("s")
    wid = cid * NS + sid
    pltpu.sync_copy(wf_hbm, wf_v)
    pltpu.sync_copy(bf_hbm, bf_v)
    lane = lax.iota(jnp.int32, 16)
    bfv = bf_v[...]

    @pl.loop(0, NSB)
    def _(sb):
        pltpu.sync_copy(src_hbm.at[wid, pl.ds(sb * SB, SB)], sidx)
        pltpu.sync_copy(dst_hbm.at[wid, pl.ds(sb * SB, SB)], didx)
        pltpu.sync_copy(neg_hbm.at[wid, pl.ds(sb * SB, SB)], nidx)
        for k in range(SB):
            c1 = pltpu.async_copy(s_hbm.at[sidx.at[k]], sbuf, sem0)
            c2 = pltpu.async_copy(d_hbm.at[didx.at[k]], dpbuf, sem1)
            c3 = pltpu.async_copy(d_hbm.at[nidx.at[k]], dnbuf, sem2)
            c1.wait()
            c2.wait()
            c3.wait()

            @pl.loop(0, CH, step=16)
            def _(g):
                row = g + lane

                def body(f, carry):
                    accp, accn = carry
                    col = jnp.full((16,), f, jnp.int32)
                    sc = plsc.load_gather(sbuf, [row, col])
                    dp = plsc.load_gather(dpbuf, [row, col])
                    dn = plsc.load_gather(dnbuf, [row, col])
                    wfk = plsc.load_gather(wf_v, [col])
                    accp = accp + jnp.maximum(sc + dp, 0.0) * wfk
                    accn = accn + jnp.maximum(sc + dn, 0.0) * wfk
                    return accp, accn

                zero = jnp.zeros((16,), jnp.float32)
                accp, accn = lax.fori_loop(0, D, body, (zero, zero))
                posv = 1.0 / (1.0 + jnp.exp(-(accp + bfv)))
                negv = 1.0 / (1.0 + jnp.exp(-(accn + bfv)))
                base = (sb * SB + k) * CH + g
                pos_v[pl.ds(base, 16)] = posv
                neg_v[pl.ds(base, 16)] = negv

    pltpu.sync_copy(pos_v, pos_hbm.at[pl.ds(wid * EPT2, EPT2)])
    pltpu.sync_copy(neg_v, negout_hbm.at[pl.ds(wid * EPT2, EPT2)])


def _dinv(da):
    return lax.rsqrt(1.0 + da[0] + da[1])


def _mm_first_body(x_ref, w_ref, dg_ref, o_ref):
    dinv = _dinv(dg_ref[...])
    o_ref[...] = jnp.dot(x_ref[...], w_ref[...],
                         preferred_element_type=jnp.float32) * dinv


def _layer_body(agg_ref, y_ref, dg_ref, b_ref, g_ref, be_ref, w_ref, o_ref):
    dinv = _dinv(dg_ref[...])
    a = agg_ref[...]
    t = (a[0] + a[1] + y_ref[...]) * dinv + b_ref[...]
    x2 = jnp.maximum(t * (g_ref[...] * _BN_SCALE) + be_ref[...], 0.0)
    o_ref[...] = jnp.dot(x2, w_ref[...],
                         preferred_element_type=jnp.float32) * dinv


def _final_body(agg_ref, y_ref, dg_ref, b_ref, ws_ref, bs_ref, wd_ref,
                bd_ref, s_ref, d_ref):
    dinv = _dinv(dg_ref[...])
    a = agg_ref[...]
    z = (a[0] + a[1] + y_ref[...]) * dinv + b_ref[...]
    s_ref[...] = jnp.dot(z, ws_ref[...],
                         preferred_element_type=jnp.float32) + bs_ref[...]
    d_ref[...] = jnp.dot(z, wd_ref[...],
                         preferred_element_type=jnp.float32) + bd_ref[...]


_row_spec = pl.BlockSpec((BT, D), lambda i: (i, 0))
_w_spec = pl.BlockSpec((D, D), lambda i: (0, 0))
_vec_spec = pl.BlockSpec((1, D), lambda i: (0, 0))
_agg_spec = pl.BlockSpec((NC, BT, D), lambda i: (0, i, 0))
_row_out = jax.ShapeDtypeStruct((NP_, D), jnp.float32)
_GRID = (NP_ // BT,)


def _mm_first(x, w, dg):
    return pl.pallas_call(
        _mm_first_body, grid=_GRID,
        in_specs=[_row_spec, _w_spec, _agg_spec],
        out_specs=_row_spec, out_shape=_row_out)(x, w, dg)


def _layer(agg, y, dg, b, g, be, w):
    return pl.pallas_call(
        _layer_body, grid=_GRID,
        in_specs=[_agg_spec, _row_spec, _agg_spec, _vec_spec, _vec_spec,
                  _vec_spec, _w_spec],
        out_specs=_row_spec, out_shape=_row_out)(agg, y, dg, b, g, be, w)


def _final(agg, y, dg, b, ws, bs, wd, bd):
    return pl.pallas_call(
        _final_body, grid=_GRID,
        in_specs=[_agg_spec, _row_spec, _agg_spec, _vec_spec, _w_spec,
                  _vec_spec, _w_spec, _vec_spec],
        out_specs=[_row_spec, _row_spec],
        out_shape=[_row_out, _row_out])(agg, y, dg, b, ws, bs, wd, bd)


def _pad_edges(a, fill):
    a = a.reshape(NW, EPT)
    pad = jnp.full((NW, EPT2 - EPT), fill, jnp.int32)
    return jnp.concatenate([a, pad], axis=1).reshape(NW, NCH, CH)


def kernel(node_feat, src, dst, neg, W1, b1, W2, b2, W3, b3, g1, be1,
           g2, be2, Ws, bs, Wd, bd, Wf, bf):
    f32 = jnp.float32
    src = src.astype(jnp.int32)
    dst = dst.astype(jnp.int32)
    neg = neg.astype(jnp.int32)
    x0 = jnp.concatenate(
        [node_feat.astype(f32), jnp.zeros((NP_ - N, D), f32)], axis=0)
    ones_tab = jnp.ones((NP_, D), f32)
    zeros_blk = jnp.zeros((ZR, D), f32)
    src_p = _pad_edges(src, 0)
    dst_p = _pad_edges(dst, PAD_DST)
    neg_p = _pad_edges(neg, 0)
    b1_ = b1.reshape(1, D)
    b2_ = b2.reshape(1, D)
    b3_ = b3.reshape(1, D)
    g1_ = g1.reshape(1, D)
    g2_ = g2.reshape(1, D)
    be1_ = be1.reshape(1, D)
    be2_ = be2.reshape(1, D)
    bs_ = bs.reshape(1, D)
    bd_ = bd.reshape(1, D)
    wf = Wf.reshape(D)
    bfv = jnp.broadcast_to(bf.reshape(1), (16,))

    dg = _agg_kernel(zeros_blk, src_p, dst_p, ones_tab)
    y1 = _mm_first(x0, W1, dg)
    agg1 = _agg_kernel(zeros_blk, src_p, dst_p, y1)
    y2 = _layer(agg1, y1, dg, b1_, g1_, be1_, W2)
    agg2 = _agg_kernel(zeros_blk, src_p, dst_p, y2)
    y3 = _layer(agg2, y2, dg, b2_, g2_, be2_, W3)
    agg3 = _agg_kernel(zeros_blk, src_p, dst_p, y3)
    s, d = _final(agg3, y3, dg, b3_, Ws, bs_, Wd, bd_)
    pos, negout = _decode_kernel(s, d, src_p, dst_p, neg_p, wf, bfv)
    pos = pos.reshape(NW, EPT2)[:, :EPT].reshape(E, 1)
    negout = negout.reshape(NW, EPT2)[:, :EPT].reshape(E, 1)
    return pos, negout


# speedup vs baseline: 3.6193x; 1.1169x over previous
import dataclasses
import functools

import jax
import jax.numpy as jnp
import numpy as np
from jax import lax
from jax.experimental import pallas as pl
from jax.experimental.pallas import tpu as pltpu
from jax.experimental.pallas import tpu_sc as plsc

N = 10000
E = 320000
D = 128
NP_ = 10240
NC = 2
NS = 16
NW = NC * NS
EPT = E // NW
CH = 128
NCH = 80
SB = 16
NSB = NCH // SB
EPT2 = NCH * CH
E2 = NW * EPT2
ROWS_ACC = NP_ // NS
ZR = 128
PAD_DST = NP_ - 1

_mesh = plsc.VectorSubcoreMesh(core_axis_name="c", subcore_axis_name="s")

_cp = pltpu.CompilerParams()
if "needs_layout_passes" in pltpu.CompilerParams.__dataclass_fields__:
    _cp = dataclasses.replace(_cp, needs_layout_passes=False)

BT = 1024
_BN_SCALE = float(1.0 / np.sqrt(1.0 + 1e-5))


@functools.partial(
    pl.kernel,
    out_type=jax.ShapeDtypeStruct((NC, NP_, D), jnp.float32),
    mesh=_mesh,
    scratch_types=[
        pltpu.VMEM((SB, CH), jnp.int32),
        pltpu.VMEM((SB, CH), jnp.int32),
        pltpu.VMEM((CH, D), jnp.float32),
        pltpu.VMEM((CH, D), jnp.float32),
        pltpu.VMEM_SHARED((NP_, D), jnp.float32),
        pltpu.SemaphoreType.DMA,
        pltpu.SemaphoreType.DMA,
    ],
)
def _agg_kernel(zeros_hbm, src_hbm, dst_hbm, y_hbm, out_hbm,
                sidx, didx, buf0, buf1, acc, sem0, sem1):
    cid = lax.axis_index("c")
    sid = lax.axis_index("s")
    wid = cid * NS + sid
    for z in range(ROWS_ACC // ZR):
        pltpu.sync_copy(zeros_hbm, acc.at[pl.ds(sid * ROWS_ACC + z * ZR, ZR)])
    plsc.subcore_barrier()

    @pl.loop(0, NSB)
    def _(sb):
        pltpu.sync_copy(src_hbm.at[wid, pl.ds(sb * SB, SB)], sidx)
        pltpu.sync_copy(dst_hbm.at[wid, pl.ds(sb * SB, SB)], didx)
        pltpu.async_copy(y_hbm.at[sidx.at[0]], buf0, sem0)
        for k in range(0, SB, 2):
            pltpu.make_async_copy(y_hbm.at[sidx.at[k]], buf0, sem0).wait()
            pltpu.async_copy(y_hbm.at[sidx.at[k + 1]], buf1, sem1)
            pltpu.sync_copy(buf0, acc.at[didx.at[k]], add=True)
            pltpu.make_async_copy(y_hbm.at[sidx.at[k + 1]], buf1, sem1).wait()
            if k + 2 < SB:
                pltpu.async_copy(y_hbm.at[sidx.at[k + 2]], buf0, sem0)
            pltpu.sync_copy(buf1, acc.at[didx.at[k + 1]], add=True)

    plsc.subcore_barrier()
    pltpu.sync_copy(acc.at[pl.ds(sid * ROWS_ACC, ROWS_ACC)],
                    out_hbm.at[cid, pl.ds(sid * ROWS_ACC, ROWS_ACC)])


@functools.partial(
    pl.kernel,
    out_type=(jax.ShapeDtypeStruct((E2,), jnp.float32),
              jax.ShapeDtypeStruct((E2,), jnp.float32)),
    mesh=_mesh,
    scratch_types=[
        pltpu.VMEM((SB, CH), jnp.int32),
        pltpu.VMEM((SB, CH), jnp.int32),
        pltpu.VMEM((SB, CH), jnp.int32),
        pltpu.VMEM((CH, D), jnp.float32),
        pltpu.VMEM((CH, D), jnp.float32),
        pltpu.VMEM((CH, D), jnp.float32),
        pltpu.VMEM((CH, D), jnp.float32),
        pltpu.VMEM((CH, D), jnp.float32),
        pltpu.VMEM((CH, D), jnp.float32),
        pltpu.VMEM((D,), jnp.float32),
        pltpu.VMEM((16,), jnp.float32),
        pltpu.VMEM((EPT2,), jnp.float32),
        pltpu.VMEM((EPT2,), jnp.float32),
        pltpu.SemaphoreType.DMA,
        pltpu.SemaphoreType.DMA,
        pltpu.SemaphoreType.DMA,
        pltpu.SemaphoreType.DMA,
        pltpu.SemaphoreType.DMA,
        pltpu.SemaphoreType.DMA,
    ],
    compiler_params=_cp,
)
def _decode_kernel(s_hbm, d_hbm, src_hbm, dst_hbm, neg_hbm, wf_hbm, bf_hbm,
                   pos_hbm, negout_hbm,
                   sidx, didx, nidx, sbuf0, dpbuf0, dnbuf0, sbuf1, dpbuf1,
                   dnbuf1, wf_v, bf_v, pos_v, neg_v,
                   gs0, gp0, gn0, gs1, gp1, gn1):
    cid = lax.axis_index("c")
    sid = lax.axis_index("s")
    wid = cid * NS + sid
    pltpu.sync_copy(wf_hbm, wf_v)
    pltpu.sync_copy(bf_hbm, bf_v)
    lane = lax.iota(jnp.int32, 16)
    bfv = bf_v[...]
    bufs = ((sbuf0, dpbuf0, dnbuf0, gs0, gp0, gn0),
            (sbuf1, dpbuf1, dnbuf1, gs1, gp1, gn1))

    def start(k, b):
        sb_, dp_, dn_, s0, s1, s2 = bufs[b]
        pltpu.async_copy(s_hbm.at[sidx.at[k]], sb_, s0)
        pltpu.async_copy(d_hbm.at[didx.at[k]], dp_, s1)
        pltpu.async_copy(d_hbm.at[nidx.at[k]], dn_, s2)

    def wait(k, b):
        sb_, dp_, dn_, s0, s1, s2 = bufs[b]
        pltpu.make_async_copy(s_hbm.at[sidx.at[k]], sb_, s0).wait()
        pltpu.make_async_copy(d_hbm.at[didx.at[k]], dp_, s1).wait()
        pltpu.make_async_copy(d_hbm.at[nidx.at[k]], dn_, s2).wait()

    def compute(k_glob, b):
        sb_, dp_, dn_ = bufs[b][:3]

        @pl.loop(0, CH, step=16)
        def _(g):
            row = g + lane
            accp = jnp.zeros((16,), jnp.float32)
            accn = jnp.zeros((16,), jnp.float32)
            for fb in range(D // 16):
                wfv = wf_v[pl.ds(fb * 16, 16)]
                for i in range(16):
                    f = fb * 16 + i
                    col = jnp.full((16,), f, jnp.int32)
                    sc = plsc.load_gather(sb_, [row, col])
                    dp = plsc.load_gather(dp_, [row, col])
                    dn = plsc.load_gather(dn_, [row, col])
                    wfk = wfv[i]
                    accp = accp + jnp.maximum(sc + dp, 0.0) * wfk
                    accn = accn + jnp.maximum(sc + dn, 0.0) * wfk
            posv = 1.0 / (1.0 + jnp.exp(-(accp + bfv)))
            negv = 1.0 / (1.0 + jnp.exp(-(accn + bfv)))
            base = k_glob * CH + g
            pos_v[pl.ds(base, 16)] = posv
            neg_v[pl.ds(base, 16)] = negv

    @pl.loop(0, NSB)
    def _(sb):
        pltpu.sync_copy(src_hbm.at[wid, pl.ds(sb * SB, SB)], sidx)
        pltpu.sync_copy(dst_hbm.at[wid, pl.ds(sb * SB, SB)], didx)
        pltpu.sync_copy(neg_hbm.at[wid, pl.ds(sb * SB, SB)], nidx)
        start(0, 0)
        for k in range(0, SB, 2):
            wait(k, 0)
            start(k + 1, 1)
            compute(sb * SB + k, 0)
            wait(k + 1, 1)
            if k + 2 < SB:
                start(k + 2, 0)
            compute(sb * SB + k + 1, 1)

    pltpu.sync_copy(pos_v, pos_hbm.at[pl.ds(wid * EPT2, EPT2)])
    pltpu.sync_copy(neg_v, negout_hbm.at[pl.ds(wid * EPT2, EPT2)])


def _dinv(da):
    return lax.rsqrt(1.0 + da[0] + da[1])


def _mm_first_body(x_ref, w_ref, dg_ref, o_ref):
    dinv = _dinv(dg_ref[...])
    o_ref[...] = jnp.dot(x_ref[...], w_ref[...],
                         preferred_element_type=jnp.float32) * dinv


def _layer_body(agg_ref, y_ref, dg_ref, b_ref, g_ref, be_ref, w_ref, o_ref):
    dinv = _dinv(dg_ref[...])
    a = agg_ref[...]
    t = (a[0] + a[1] + y_ref[...]) * dinv + b_ref[...]
    x2 = jnp.maximum(t * (g_ref[...] * _BN_SCALE) + be_ref[...], 0.0)
    o_ref[...] = jnp.dot(x2, w_ref[...],
                         preferred_element_type=jnp.float32) * dinv


def _final_body(agg_ref, y_ref, dg_ref, b_ref, ws_ref, bs_ref, wd_ref,
                bd_ref, s_ref, d_ref):
    dinv = _dinv(dg_ref[...])
    a = agg_ref[...]
    z = (a[0] + a[1] + y_ref[...]) * dinv + b_ref[...]
    s_ref[...] = jnp.dot(z, ws_ref[...],
                         preferred_element_type=jnp.float32) + bs_ref[...]
    d_ref[...] = jnp.dot(z, wd_ref[...],
                         preferred_element_type=jnp.float32) + bd_ref[...]


_row_spec = pl.BlockSpec((BT, D), lambda i: (i, 0))
_w_spec = pl.BlockSpec((D, D), lambda i: (0, 0))
_vec_spec = pl.BlockSpec((1, D), lambda i: (0, 0))
_agg_spec = pl.BlockSpec((NC, BT, D), lambda i: (0, i, 0))
_row_out = jax.ShapeDtypeStruct((NP_, D), jnp.float32)
_GRID = (NP_ // BT,)


def _mm_first(x, w, dg):
    return pl.pallas_call(
        _mm_first_body, grid=_GRID,
        in_specs=[_row_spec, _w_spec, _agg_spec],
        out_specs=_row_spec, out_shape=_row_out)(x, w, dg)


def _layer(agg, y, dg, b, g, be, w):
    return pl.pallas_call(
        _layer_body, grid=_GRID,
        in_specs=[_agg_spec, _row_spec, _agg_spec, _vec_spec, _vec_spec,
                  _vec_spec, _w_spec],
        out_specs=_row_spec, out_shape=_row_out)(agg, y, dg, b, g, be, w)


def _final(agg, y, dg, b, ws, bs, wd, bd):
    return pl.pallas_call(
        _final_body, grid=_GRID,
        in_specs=[_agg_spec, _row_spec, _agg_spec, _vec_spec, _w_spec,
                  _vec_spec, _w_spec, _vec_spec],
        out_specs=[_row_spec, _row_spec],
        out_shape=[_row_out, _row_out])(agg, y, dg, b, ws, bs, wd, bd)


def _pad_edges(a, fill):
    a = a.reshape(NW, EPT)
    pad = jnp.full((NW, EPT2 - EPT), fill, jnp.int32)
    return jnp.concatenate([a, pad], axis=1).reshape(NW, NCH, CH)


def kernel(node_feat, src, dst, neg, W1, b1, W2, b2, W3, b3, g1, be1,
           g2, be2, Ws, bs, Wd, bd, Wf, bf):
    f32 = jnp.float32
    src = src.astype(jnp.int32)
    dst = dst.astype(jnp.int32)
    neg = neg.astype(jnp.int32)
    x0 = jnp.concatenate(
        [node_feat.astype(f32), jnp.zeros((NP_ - N, D), f32)], axis=0)
    ones_tab = jnp.ones((NP_, D), f32)
    zeros_blk = jnp.zeros((ZR, D), f32)
    src_p = _pad_edges(src, 0)
    dst_p = _pad_edges(dst, PAD_DST)
    neg_p = _pad_edges(neg, 0)
    b1_ = b1.reshape(1, D)
    b2_ = b2.reshape(1, D)
    b3_ = b3.reshape(1, D)
    g1_ = g1.reshape(1, D)
    g2_ = g2.reshape(1, D)
    be1_ = be1.reshape(1, D)
    be2_ = be2.reshape(1, D)
    bs_ = bs.reshape(1, D)
    bd_ = bd.reshape(1, D)
    wf = Wf.reshape(D)
    bfv = jnp.broadcast_to(bf.reshape(1), (16,))

    dg = _agg_kernel(zeros_blk, src_p, dst_p, ones_tab)
    y1 = _mm_first(x0, W1, dg)
    agg1 = _agg_kernel(zeros_blk, src_p, dst_p, y1)
    y2 = _layer(agg1, y1, dg, b1_, g1_, be1_, W2)
    agg2 = _agg_kernel(zeros_blk, src_p, dst_p, y2)
    y3 = _layer(agg2, y2, dg, b2_, g2_, be2_, W3)
    agg3 = _agg_kernel(zeros_blk, src_p, dst_p, y3)
    s, d = _final(agg3, y3, dg, b3_, Ws, bs_, Wd, bd_)
    pos, negout = _decode_kernel(s, d, src_p, dst_p, neg_p, wf, bfv)
    pos = pos.reshape(NW, EPT2)[:, :EPT].reshape(E, 1)
    negout = negout.reshape(NW, EPT2)[:, :EPT].reshape(E, 1)
    return pos, negout


# speedup vs baseline: 5.8015x; 1.6030x over previous
import dataclasses
import functools

import jax
import jax.numpy as jnp
import numpy as np
from jax import lax
from jax.experimental import pallas as pl
from jax.experimental.pallas import tpu as pltpu
from jax.experimental.pallas import tpu_sc as plsc

N = 10000
E = 320000
D = 128
NP_ = 10240
NC = 2
NS = 16
NW = NC * NS
EPT = E // NW
CH = 128
NCH = 80
SB = 16
NSB = NCH // SB
EPT2 = NCH * CH
E2 = NW * EPT2
ROWS_ACC = NP_ // NS
ZR = 128
PAD_DST = NP_ - 1

_mesh = plsc.VectorSubcoreMesh(core_axis_name="c", subcore_axis_name="s")

_cp = pltpu.CompilerParams()
if "needs_layout_passes" in pltpu.CompilerParams.__dataclass_fields__:
    _cp = dataclasses.replace(_cp, needs_layout_passes=False)

BT = 1024
_BN_SCALE = float(1.0 / np.sqrt(1.0 + 1e-5))


@functools.partial(
    pl.kernel,
    out_type=jax.ShapeDtypeStruct((NC, NP_, D), jnp.float32),
    mesh=_mesh,
    scratch_types=[
        pltpu.VMEM((SB, CH), jnp.int32),
        pltpu.VMEM((SB, CH), jnp.int32),
        pltpu.VMEM((CH, D), jnp.float32),
        pltpu.VMEM((CH, D), jnp.float32),
        pltpu.VMEM_SHARED((NP_, D), jnp.float32),
        pltpu.SemaphoreType.DMA,
        pltpu.SemaphoreType.DMA,
    ],
)
def _agg_kernel(zeros_hbm, src_hbm, dst_hbm, y_hbm, out_hbm,
                sidx, didx, buf0, buf1, acc, sem0, sem1):
    cid = lax.axis_index("c")
    sid = lax.axis_index("s")
    wid = cid * NS + sid
    for z in range(ROWS_ACC // ZR):
        pltpu.sync_copy(zeros_hbm, acc.at[pl.ds(sid * ROWS_ACC + z * ZR, ZR)])
    plsc.subcore_barrier()

    @pl.loop(0, NSB)
    def _(sb):
        pltpu.sync_copy(src_hbm.at[wid, pl.ds(sb * SB, SB)], sidx)
        pltpu.sync_copy(dst_hbm.at[wid, pl.ds(sb * SB, SB)], didx)
        pltpu.async_copy(y_hbm.at[sidx.at[0]], buf0, sem0)
        for k in range(0, SB, 2):
            pltpu.make_async_copy(y_hbm.at[sidx.at[k]], buf0, sem0).wait()
            pltpu.async_copy(y_hbm.at[sidx.at[k + 1]], buf1, sem1)
            pltpu.sync_copy(buf0, acc.at[didx.at[k]], add=True)
            pltpu.make_async_copy(y_hbm.at[sidx.at[k + 1]], buf1, sem1).wait()
            if k + 2 < SB:
                pltpu.async_copy(y_hbm.at[sidx.at[k + 2]], buf0, sem0)
            pltpu.sync_copy(buf1, acc.at[didx.at[k + 1]], add=True)

    plsc.subcore_barrier()
    pltpu.sync_copy(acc.at[pl.ds(sid * ROWS_ACC, ROWS_ACC)],
                    out_hbm.at[cid, pl.ds(sid * ROWS_ACC, ROWS_ACC)])


@functools.partial(
    pl.kernel,
    out_type=(jax.ShapeDtypeStruct((E2,), jnp.float32),
              jax.ShapeDtypeStruct((E2,), jnp.float32)),
    mesh=_mesh,
    scratch_types=[
        pltpu.VMEM((SB, CH), jnp.int32),
        pltpu.VMEM((SB, CH), jnp.int32),
        pltpu.VMEM((SB, CH), jnp.int32),
        pltpu.VMEM((CH, D), jnp.float32),
        pltpu.VMEM((CH, D), jnp.float32),
        pltpu.VMEM((CH, D), jnp.float32),
        pltpu.VMEM((CH, D), jnp.float32),
        pltpu.VMEM((CH, D), jnp.float32),
        pltpu.VMEM((CH, D), jnp.float32),
        pltpu.VMEM((D,), jnp.float32),
        pltpu.VMEM((16,), jnp.float32),
        pltpu.VMEM((EPT2,), jnp.float32),
        pltpu.VMEM((EPT2,), jnp.float32),
        pltpu.SemaphoreType.DMA,
        pltpu.SemaphoreType.DMA,
        pltpu.SemaphoreType.DMA,
        pltpu.SemaphoreType.DMA,
        pltpu.SemaphoreType.DMA,
        pltpu.SemaphoreType.DMA,
    ],
    compiler_params=_cp,
)
def _decode_kernel(s_hbm, d_hbm, src_hbm, dst_hbm, neg_hbm, wf_hbm, bf_hbm,
                   pos_hbm, negout_hbm,
                   sidx, didx, nidx, sbuf0, dpbuf0, dnbuf0, sbuf1, dpbuf1,
                   dnbuf1, wf_v, bf_v, pos_v, neg_v,
                   gs0, gp0, gn0, gs1, gp1, gn1):
    cid = lax.axis_index("c")
    sid = lax.axis_index("s")
    wid = cid * NS + sid
    pltpu.sync_copy(wf_hbm, wf_v)
    pltpu.sync_copy(bf_hbm, bf_v)
    lane = lax.iota(jnp.int32, 16)
    bfv = bf_v[...]
    bufs = ((sbuf0, dpbuf0, dnbuf0, gs0, gp0, gn0),
            (sbuf1, dpbuf1, dnbuf1, gs1, gp1, gn1))

    def start(k, b):
        sb_, dp_, dn_, s0, s1, s2 = bufs[b]
        pltpu.async_copy(s_hbm.at[sidx.at[k]], sb_, s0)
        pltpu.async_copy(d_hbm.at[didx.at[k]], dp_, s1)
        pltpu.async_copy(d_hbm.at[nidx.at[k]], dn_, s2)

    def wait(k, b):
        sb_, dp_, dn_, s0, s1, s2 = bufs[b]
        pltpu.make_async_copy(s_hbm.at[sidx.at[k]], sb_, s0).wait()
        pltpu.make_async_copy(d_hbm.at[didx.at[k]], dp_, s1).wait()
        pltpu.make_async_copy(d_hbm.at[nidx.at[k]], dn_, s2).wait()

    wfc = [wf_v[pl.ds(fb * 16, 16)] for fb in range(D // 16)]
    lastlane = lane == 15

    def compute(k_glob, b):
        sb_, dp_, dn_ = bufs[b][:3]

        @pl.loop(0, CH, step=2)
        def _(g):
            base = k_glob * CH + g
            for e in range(2):
                r = g + e
                accp = jnp.zeros((16,), jnp.float32)
                accn = jnp.zeros((16,), jnp.float32)
                for fb in range(D // 16):
                    cs = pl.ds(fb * 16, 16)
                    sc = sb_[r, cs]
                    dp = dp_[r, cs]
                    dn = dn_[r, cs]
                    accp = accp + jnp.maximum(sc + dp, 0.0) * wfc[fb]
                    accn = accn + jnp.maximum(sc + dn, 0.0) * wfc[fb]
                tgt = jnp.full((16,), base + e, jnp.int32)
                plsc.store_scatter(pos_v, [tgt], plsc.cumsum(accp),
                                   mask=lastlane)
                plsc.store_scatter(neg_v, [tgt], plsc.cumsum(accn),
                                   mask=lastlane)

    @pl.loop(0, NSB)
    def _(sb):
        pltpu.sync_copy(src_hbm.at[wid, pl.ds(sb * SB, SB)], sidx)
        pltpu.sync_copy(dst_hbm.at[wid, pl.ds(sb * SB, SB)], didx)
        pltpu.sync_copy(neg_hbm.at[wid, pl.ds(sb * SB, SB)], nidx)
        start(0, 0)
        for k in range(0, SB, 2):
            wait(k, 0)
            start(k + 1, 1)
            compute(sb * SB + k, 0)
            wait(k + 1, 1)
            if k + 2 < SB:
                start(k + 2, 0)
            compute(sb * SB + k + 1, 1)

    @pl.loop(0, EPT2, step=16)
    def _(i):
        sl = pl.ds(i, 16)
        pos_v[sl] = 1.0 / (1.0 + jnp.exp(-(pos_v[sl] + bfv)))
        neg_v[sl] = 1.0 / (1.0 + jnp.exp(-(neg_v[sl] + bfv)))

    pltpu.sync_copy(pos_v, pos_hbm.at[pl.ds(wid * EPT2, EPT2)])
    pltpu.sync_copy(neg_v, negout_hbm.at[pl.ds(wid * EPT2, EPT2)])


def _dinv(da):
    return lax.rsqrt(1.0 + da[0] + da[1])


def _mm_first_body(x_ref, w_ref, dg_ref, o_ref):
    dinv = _dinv(dg_ref[...])
    o_ref[...] = jnp.dot(x_ref[...], w_ref[...],
                         preferred_element_type=jnp.float32) * dinv


def _layer_body(agg_ref, y_ref, dg_ref, b_ref, g_ref, be_ref, w_ref, o_ref):
    dinv = _dinv(dg_ref[...])
    a = agg_ref[...]
    t = (a[0] + a[1] + y_ref[...]) * dinv + b_ref[...]
    x2 = jnp.maximum(t * (g_ref[...] * _BN_SCALE) + be_ref[...], 0.0)
    o_ref[...] = jnp.dot(x2, w_ref[...],
                         preferred_element_type=jnp.float32) * dinv


def _final_body(agg_ref, y_ref, dg_ref, b_ref, ws_ref, bs_ref, wd_ref,
                bd_ref, s_ref, d_ref):
    dinv = _dinv(dg_ref[...])
    a = agg_ref[...]
    z = (a[0] + a[1] + y_ref[...]) * dinv + b_ref[...]
    s_ref[...] = jnp.dot(z, ws_ref[...],
                         preferred_element_type=jnp.float32) + bs_ref[...]
    d_ref[...] = jnp.dot(z, wd_ref[...],
                         preferred_element_type=jnp.float32) + bd_ref[...]


_row_spec = pl.BlockSpec((BT, D), lambda i: (i, 0))
_w_spec = pl.BlockSpec((D, D), lambda i: (0, 0))
_vec_spec = pl.BlockSpec((1, D), lambda i: (0, 0))
_agg_spec = pl.BlockSpec((NC, BT, D), lambda i: (0, i, 0))
_row_out = jax.ShapeDtypeStruct((NP_, D), jnp.float32)
_GRID = (NP_ // BT,)


def _mm_first(x, w, dg):
    return pl.pallas_call(
        _mm_first_body, grid=_GRID,
        in_specs=[_row_spec, _w_spec, _agg_spec],
        out_specs=_row_spec, out_shape=_row_out)(x, w, dg)


def _layer(agg, y, dg, b, g, be, w):
    return pl.pallas_call(
        _layer_body, grid=_GRID,
        in_specs=[_agg_spec, _row_spec, _agg_spec, _vec_spec, _vec_spec,
                  _vec_spec, _w_spec],
        out_specs=_row_spec, out_shape=_row_out)(agg, y, dg, b, g, be, w)


def _final(agg, y, dg, b, ws, bs, wd, bd):
    return pl.pallas_call(
        _final_body, grid=_GRID,
        in_specs=[_agg_spec, _row_spec, _agg_spec, _vec_spec, _w_spec,
                  _vec_spec, _w_spec, _vec_spec],
        out_specs=[_row_spec, _row_spec],
        out_shape=[_row_out, _row_out])(agg, y, dg, b, ws, bs, wd, bd)


def _pad_edges(a, fill):
    a = a.reshape(NW, EPT)
    pad = jnp.full((NW, EPT2 - EPT), fill, jnp.int32)
    return jnp.concatenate([a, pad], axis=1).reshape(NW, NCH, CH)


def kernel(node_feat, src, dst, neg, W1, b1, W2, b2, W3, b3, g1, be1,
           g2, be2, Ws, bs, Wd, bd, Wf, bf):
    f32 = jnp.float32
    src = src.astype(jnp.int32)
    dst = dst.astype(jnp.int32)
    neg = neg.astype(jnp.int32)
    x0 = jnp.concatenate(
        [node_feat.astype(f32), jnp.zeros((NP_ - N, D), f32)], axis=0)
    ones_tab = jnp.ones((NP_, D), f32)
    zeros_blk = jnp.zeros((ZR, D), f32)
    src_p = _pad_edges(src, 0)
    dst_p = _pad_edges(dst, PAD_DST)
    neg_p = _pad_edges(neg, 0)
    b1_ = b1.reshape(1, D)
    b2_ = b2.reshape(1, D)
    b3_ = b3.reshape(1, D)
    g1_ = g1.reshape(1, D)
    g2_ = g2.reshape(1, D)
    be1_ = be1.reshape(1, D)
    be2_ = be2.reshape(1, D)
    bs_ = bs.reshape(1, D)
    bd_ = bd.reshape(1, D)
    wf = Wf.reshape(D)
    bfv = jnp.broadcast_to(bf.reshape(1), (16,))

    dg = _agg_kernel(zeros_blk, src_p, dst_p, ones_tab)
    y1 = _mm_first(x0, W1, dg)
    agg1 = _agg_kernel(zeros_blk, src_p, dst_p, y1)
    y2 = _layer(agg1, y1, dg, b1_, g1_, be1_, W2)
    agg2 = _agg_kernel(zeros_blk, src_p, dst_p, y2)
    y3 = _layer(agg2, y2, dg, b2_, g2_, be2_, W3)
    agg3 = _agg_kernel(zeros_blk, src_p, dst_p, y3)
    s, d = _final(agg3, y3, dg, b3_, Ws, bs_, Wd, bd_)
    pos, negout = _decode_kernel(s, d, src_p, dst_p, neg_p, wf, bfv)
    pos = pos.reshape(NW, EPT2)[:, :EPT].reshape(E, 1)
    negout = negout.reshape(NW, EPT2)[:, :EPT].reshape(E, 1)
    return pos, negout


# speedup vs baseline: 7.0054x; 1.2075x over previous
import dataclasses
import functools

import jax
import jax.numpy as jnp
import numpy as np
from jax import lax
from jax.experimental import pallas as pl
from jax.experimental.pallas import tpu as pltpu
from jax.experimental.pallas import tpu_sc as plsc

N = 10000
E = 320000
D = 128
NP_ = 10240
NC = 2
NS = 16
NW = NC * NS
EPT = E // NW
CH = 128
NCH = 80
SB = 16
NSB = NCH // SB
EPT2 = NCH * CH
E2 = NW * EPT2
ROWS_ACC = NP_ // NS
ZR = 128
PAD_DST = NP_ - 1

_mesh = plsc.VectorSubcoreMesh(core_axis_name="c", subcore_axis_name="s")

_cp = pltpu.CompilerParams()
if "needs_layout_passes" in pltpu.CompilerParams.__dataclass_fields__:
    _cp = dataclasses.replace(_cp, needs_layout_passes=False)

BT = 1024
_BN_SCALE = float(1.0 / np.sqrt(1.0 + 1e-5))


@functools.partial(
    pl.kernel,
    out_type=jax.ShapeDtypeStruct((NW, NP_), jnp.float32),
    mesh=_mesh,
    scratch_types=[
        pltpu.VMEM((SB, CH), jnp.int32),
        pltpu.VMEM((NP_,), jnp.float32),
    ],
    compiler_params=_cp,
)
def _deg_kernel(dst_hbm, out_hbm, didx, hist):
    cid = lax.axis_index("c")
    sid = lax.axis_index("s")
    wid = cid * NS + sid

    @pl.loop(0, NP_, step=16)
    def _(i):
        hist[pl.ds(i, 16)] = jnp.zeros((16,), jnp.float32)

    ones = jnp.ones((16,), jnp.float32)

    @pl.loop(0, NSB)
    def _(sb):
        pltpu.sync_copy(dst_hbm.at[wid, pl.ds(sb * SB, SB)], didx)

        @pl.loop(0, SB)
        def _(k):
            for g in range(CH // 16):
                idx = didx[k, pl.ds(g * 16, 16)]
                plsc.addupdate_scatter(hist, [idx], ones)

    pltpu.sync_copy(hist, out_hbm.at[wid])


@functools.partial(
    pl.kernel,
    out_type=jax.ShapeDtypeStruct((NC, NP_, D), jnp.float32),
    mesh=_mesh,
    scratch_types=[
        pltpu.VMEM((SB, CH), jnp.int32),
        pltpu.VMEM((SB, CH), jnp.int32),
        pltpu.VMEM((CH, D), jnp.float32),
        pltpu.VMEM((CH, D), jnp.float32),
        pltpu.VMEM_SHARED((NP_, D), jnp.float32),
        pltpu.SemaphoreType.DMA,
        pltpu.SemaphoreType.DMA,
    ],
)
def _agg_kernel(zeros_hbm, src_hbm, dst_hbm, y_hbm, out_hbm,
                sidx, didx, buf0, buf1, acc, sem0, sem1):
    cid = lax.axis_index("c")
    sid = lax.axis_index("s")
    wid = cid * NS + sid
    for z in range(ROWS_ACC // ZR):
        pltpu.sync_copy(zeros_hbm, acc.at[pl.ds(sid * ROWS_ACC + z * ZR, ZR)])
    plsc.subcore_barrier()

    @pl.loop(0, NSB)
    def _(sb):
        pltpu.sync_copy(src_hbm.at[wid, pl.ds(sb * SB, SB)], sidx)
        pltpu.sync_copy(dst_hbm.at[wid, pl.ds(sb * SB, SB)], didx)
        pltpu.async_copy(y_hbm.at[sidx.at[0]], buf0, sem0)
        for k in range(0, SB, 2):
            pltpu.make_async_copy(y_hbm.at[sidx.at[k]], buf0, sem0).wait()
            pltpu.async_copy(y_hbm.at[sidx.at[k + 1]], buf1, sem1)
            pltpu.sync_copy(buf0, acc.at[didx.at[k]], add=True)
            pltpu.make_async_copy(y_hbm.at[sidx.at[k + 1]], buf1, sem1).wait()
            if k + 2 < SB:
                pltpu.async_copy(y_hbm.at[sidx.at[k + 2]], buf0, sem0)
            pltpu.sync_copy(buf1, acc.at[didx.at[k + 1]], add=True)

    plsc.subcore_barrier()
    pltpu.sync_copy(acc.at[pl.ds(sid * ROWS_ACC, ROWS_ACC)],
                    out_hbm.at[cid, pl.ds(sid * ROWS_ACC, ROWS_ACC)])


@functools.partial(
    pl.kernel,
    out_type=(jax.ShapeDtypeStruct((E2,), jnp.float32),
              jax.ShapeDtypeStruct((E2,), jnp.float32)),
    mesh=_mesh,
    scratch_types=[
        pltpu.VMEM((SB, CH), jnp.int32),
        pltpu.VMEM((SB, CH), jnp.int32),
        pltpu.VMEM((SB, CH), jnp.int32),
        pltpu.VMEM((CH, D), jnp.float32),
        pltpu.VMEM((CH, D), jnp.float32),
        pltpu.VMEM((CH, D), jnp.float32),
        pltpu.VMEM((CH, D), jnp.float32),
        pltpu.VMEM((CH, D), jnp.float32),
        pltpu.VMEM((CH, D), jnp.float32),
        pltpu.VMEM((D,), jnp.float32),
        pltpu.VMEM((16,), jnp.float32),
        pltpu.VMEM((EPT2,), jnp.float32),
        pltpu.VMEM((EPT2,), jnp.float32),
        pltpu.SemaphoreType.DMA,
        pltpu.SemaphoreType.DMA,
        pltpu.SemaphoreType.DMA,
        pltpu.SemaphoreType.DMA,
        pltpu.SemaphoreType.DMA,
        pltpu.SemaphoreType.DMA,
    ],
    compiler_params=_cp,
)
def _decode_kernel(s_hbm, d_hbm, src_hbm, dst_hbm, neg_hbm, wf_hbm, bf_hbm,
                   pos_hbm, negout_hbm,
                   sidx, didx, nidx, sbuf0, dpbuf0, dnbuf0, sbuf1, dpbuf1,
                   dnbuf1, wf_v, bf_v, pos_v, neg_v,
                   gs0, gp0, gn0, gs1, gp1, gn1):
    cid = lax.axis_index("c")
    sid = lax.axis_index("s")
    wid = cid * NS + sid
    pltpu.sync_copy(wf_hbm, wf_v)
    pltpu.sync_copy(bf_hbm, bf_v)
    lane = lax.iota(jnp.int32, 16)
    bfv = bf_v[...]
    bufs = ((sbuf0, dpbuf0, dnbuf0, gs0, gp0, gn0),
            (sbuf1, dpbuf1, dnbuf1, gs1, gp1, gn1))

    def start(k, b):
        sb_, dp_, dn_, s0, s1, s2 = bufs[b]
        pltpu.async_copy(s_hbm.at[sidx.at[k]], sb_, s0)
        pltpu.async_copy(d_hbm.at[didx.at[k]], dp_, s1)
        pltpu.async_copy(d_hbm.at[nidx.at[k]], dn_, s2)

    def wait(k, b):
        sb_, dp_, dn_, s0, s1, s2 = bufs[b]
        pltpu.make_async_copy(s_hbm.at[sidx.at[k]], sb_, s0).wait()
        pltpu.make_async_copy(d_hbm.at[didx.at[k]], dp_, s1).wait()
        pltpu.make_async_copy(d_hbm.at[nidx.at[k]], dn_, s2).wait()

    wfc = [wf_v[pl.ds(fb * 16, 16)] for fb in range(D // 16)]
    lastlane = lane == 15

    def compute(k_glob, b):
        sb_, dp_, dn_ = bufs[b][:3]

        @pl.loop(0, CH, step=2)
        def _(g):
            base = k_glob * CH + g
            for e in range(2):
                r = g + e
                accp = jnp.zeros((16,), jnp.float32)
                accn = jnp.zeros((16,), jnp.float32)
                for fb in range(D // 16):
                    cs = pl.ds(fb * 16, 16)
                    sc = sb_[r, cs]
                    dp = dp_[r, cs]
                    dn = dn_[r, cs]
                    accp = accp + jnp.maximum(sc + dp, 0.0) * wfc[fb]
                    accn = accn + jnp.maximum(sc + dn, 0.0) * wfc[fb]
                tgt = jnp.full((16,), base + e, jnp.int32)
                plsc.store_scatter(pos_v, [tgt], plsc.cumsum(accp),
                                   mask=lastlane)
                plsc.store_scatter(neg_v, [tgt], plsc.cumsum(accn),
                                   mask=lastlane)

    @pl.loop(0, NSB)
    def _(sb):
        pltpu.sync_copy(src_hbm.at[wid, pl.ds(sb * SB, SB)], sidx)
        pltpu.sync_copy(dst_hbm.at[wid, pl.ds(sb * SB, SB)], didx)
        pltpu.sync_copy(neg_hbm.at[wid, pl.ds(sb * SB, SB)], nidx)
        start(0, 0)
        for k in range(0, SB, 2):
            wait(k, 0)
            start(k + 1, 1)
            compute(sb * SB + k, 0)
            wait(k + 1, 1)
            if k + 2 < SB:
                start(k + 2, 0)
            compute(sb * SB + k + 1, 1)

    @pl.loop(0, EPT2, step=16)
    def _(i):
        sl = pl.ds(i, 16)
        pos_v[sl] = 1.0 / (1.0 + jnp.exp(-(pos_v[sl] + bfv)))
        neg_v[sl] = 1.0 / (1.0 + jnp.exp(-(neg_v[sl] + bfv)))

    pltpu.sync_copy(pos_v, pos_hbm.at[pl.ds(wid * EPT2, EPT2)])
    pltpu.sync_copy(neg_v, negout_hbm.at[pl.ds(wid * EPT2, EPT2)])


def _dinv(h_ref):
    deg = lax.dot_general(h_ref[...], jnp.ones((NW, D), jnp.float32),
                          (((0,), (0,)), ((), ())),
                          preferred_element_type=jnp.float32)
    return lax.rsqrt(1.0 + deg)


def _mm_first_body(x_ref, w_ref, dg_ref, o_ref):
    dinv = _dinv(dg_ref)
    o_ref[...] = jnp.dot(x_ref[...], w_ref[...],
                         preferred_element_type=jnp.float32) * dinv


def _layer_body(agg_ref, y_ref, dg_ref, b_ref, g_ref, be_ref, w_ref, o_ref):
    dinv = _dinv(dg_ref)
    a = agg_ref[...]
    t = (a[0] + a[1] + y_ref[...]) * dinv + b_ref[...]
    x2 = jnp.maximum(t * (g_ref[...] * _BN_SCALE) + be_ref[...], 0.0)
    o_ref[...] = jnp.dot(x2, w_ref[...],
                         preferred_element_type=jnp.float32) * dinv


def _final_body(agg_ref, y_ref, dg_ref, b_ref, ws_ref, bs_ref, wd_ref,
                bd_ref, s_ref, d_ref):
    dinv = _dinv(dg_ref)
    a = agg_ref[...]
    z = (a[0] + a[1] + y_ref[...]) * dinv + b_ref[...]
    s_ref[...] = jnp.dot(z, ws_ref[...],
                         preferred_element_type=jnp.float32) + bs_ref[...]
    d_ref[...] = jnp.dot(z, wd_ref[...],
                         preferred_element_type=jnp.float32) + bd_ref[...]


_row_spec = pl.BlockSpec((BT, D), lambda i: (i, 0))
_w_spec = pl.BlockSpec((D, D), lambda i: (0, 0))
_vec_spec = pl.BlockSpec((1, D), lambda i: (0, 0))
_agg_spec = pl.BlockSpec((NC, BT, D), lambda i: (0, i, 0))
_hist_spec = pl.BlockSpec((NW, BT), lambda i: (0, i))
_row_out = jax.ShapeDtypeStruct((NP_, D), jnp.float32)
_GRID = (NP_ // BT,)


def _mm_first(x, w, dg):
    return pl.pallas_call(
        _mm_first_body, grid=_GRID,
        in_specs=[_row_spec, _w_spec, _hist_spec],
        out_specs=_row_spec, out_shape=_row_out)(x, w, dg)


def _layer(agg, y, dg, b, g, be, w):
    return pl.pallas_call(
        _layer_body, grid=_GRID,
        in_specs=[_agg_spec, _row_spec, _hist_spec, _vec_spec, _vec_spec,
                  _vec_spec, _w_spec],
        out_specs=_row_spec, out_shape=_row_out)(agg, y, dg, b, g, be, w)


def _final(agg, y, dg, b, ws, bs, wd, bd):
    return pl.pallas_call(
        _final_body, grid=_GRID,
        in_specs=[_agg_spec, _row_spec, _hist_spec, _vec_spec, _w_spec,
                  _vec_spec, _w_spec, _vec_spec],
        out_specs=[_row_spec, _row_spec],
        out_shape=[_row_out, _row_out])(agg, y, dg, b, ws, bs, wd, bd)


def _pad_edges(a, fill):
    a = a.reshape(NW, EPT)
    pad = jnp.full((NW, EPT2 - EPT), fill, jnp.int32)
    return jnp.concatenate([a, pad], axis=1).reshape(NW, NCH, CH)


def kernel(node_feat, src, dst, neg, W1, b1, W2, b2, W3, b3, g1, be1,
           g2, be2, Ws, bs, Wd, bd, Wf, bf):
    f32 = jnp.float32
    src = src.astype(jnp.int32)
    dst = dst.astype(jnp.int32)
    neg = neg.astype(jnp.int32)
    x0 = jnp.concatenate(
        [node_feat.astype(f32), jnp.zeros((NP_ - N, D), f32)], axis=0)
    zeros_blk = jnp.zeros((ZR, D), f32)
    src_p = _pad_edges(src, 0)
    dst_p = _pad_edges(dst, PAD_DST)
    neg_p = _pad_edges(neg, 0)
    b1_ = b1.reshape(1, D)
    b2_ = b2.reshape(1, D)
    b3_ = b3.reshape(1, D)
    g1_ = g1.reshape(1, D)
    g2_ = g2.reshape(1, D)
    be1_ = be1.reshape(1, D)
    be2_ = be2.reshape(1, D)
    bs_ = bs.reshape(1, D)
    bd_ = bd.reshape(1, D)
    wf = Wf.reshape(D)
    bfv = jnp.broadcast_to(bf.reshape(1), (16,))

    dg = _deg_kernel(dst_p)
    y1 = _mm_first(x0, W1, dg)
    agg1 = _agg_kernel(zeros_blk, src_p, dst_p, y1)
    y2 = _layer(agg1, y1, dg, b1_, g1_, be1_, W2)
    agg2 = _agg_kernel(zeros_blk, src_p, dst_p, y2)
    y3 = _layer(agg2, y2, dg, b2_, g2_, be2_, W3)
    agg3 = _agg_kernel(zeros_blk, src_p, dst_p, y3)
    s, d = _final(agg3, y3, dg, b3_, Ws, bs_, Wd, bd_)
    pos, negout = _decode_kernel(s, d, src_p, dst_p, neg_p, wf, bfv)
    pos = pos.reshape(NW, EPT2)[:, :EPT].reshape(E, 1)
    negout = negout.reshape(NW, EPT2)[:, :EPT].reshape(E, 1)
    return pos, negout


# speedup vs baseline: 7.1584x; 1.0218x over previous
import dataclasses
import functools

import jax
import jax.numpy as jnp
import numpy as np
from jax import lax
from jax.experimental import pallas as pl
from jax.experimental.pallas import tpu as pltpu
from jax.experimental.pallas import tpu_sc as plsc

N = 10000
E = 320000
D = 128
NP_ = 10240
NC = 2
NS = 16
NW = NC * NS
EPT = E // NW
CH = 128
NCH = 80
SB = 16
NSB = NCH // SB
EPT2 = NCH * CH
E2 = NW * EPT2
ROWS_ACC = NP_ // NS
ZR = 128
PAD_DST = NP_ - 1
CH2 = 64
NCH2 = EPT2 // CH2
NSB2 = NCH2 // SB

_mesh = plsc.VectorSubcoreMesh(core_axis_name="c", subcore_axis_name="s")

_cp = pltpu.CompilerParams()
if "needs_layout_passes" in pltpu.CompilerParams.__dataclass_fields__:
    _cp = dataclasses.replace(_cp, needs_layout_passes=False)

BT = 1024
_BN_SCALE = float(1.0 / np.sqrt(1.0 + 1e-5))


@functools.partial(
    pl.kernel,
    out_type=jax.ShapeDtypeStruct((NW, NP_), jnp.float32),
    mesh=_mesh,
    scratch_types=[
        pltpu.VMEM((SB, CH), jnp.int32),
        pltpu.VMEM((NP_,), jnp.float32),
    ],
    compiler_params=_cp,
)
def _deg_kernel(dst_hbm, out_hbm, didx, hist):
    cid = lax.axis_index("c")
    sid = lax.axis_index("s")
    wid = cid * NS + sid

    @pl.loop(0, NP_, step=16)
    def _(i):
        hist[pl.ds(i, 16)] = jnp.zeros((16,), jnp.float32)

    ones = jnp.ones((16,), jnp.float32)

    @pl.loop(0, NSB)
    def _(sb):
        pltpu.sync_copy(dst_hbm.at[wid, pl.ds(sb * SB, SB)], didx)

        @pl.loop(0, SB)
        def _(k):
            for g in range(CH // 16):
                idx = didx[k, pl.ds(g * 16, 16)]
                plsc.addupdate_scatter(hist, [idx], ones)

    pltpu.sync_copy(hist, out_hbm.at[wid])


NBUF = 4

@functools.partial(
    pl.kernel,
    out_type=jax.ShapeDtypeStruct((NC, NP_, D), jnp.float32),
    mesh=_mesh,
    scratch_types=[
        pltpu.VMEM((SB, CH2), jnp.int32),
        pltpu.VMEM((SB, CH2), jnp.int32),
        pltpu.VMEM((CH2, D), jnp.float32),
        pltpu.VMEM((CH2, D), jnp.float32),
        pltpu.VMEM((CH2, D), jnp.float32),
        pltpu.VMEM((CH2, D), jnp.float32),
        pltpu.VMEM_SHARED((NP_, D), jnp.float32),
        pltpu.SemaphoreType.DMA,
        pltpu.SemaphoreType.DMA,
        pltpu.SemaphoreType.DMA,
        pltpu.SemaphoreType.DMA,
        pltpu.SemaphoreType.DMA,
        pltpu.SemaphoreType.DMA,
        pltpu.SemaphoreType.DMA,
        pltpu.SemaphoreType.DMA,
    ],
)
def _agg_kernel(zeros_hbm, src_hbm, dst_hbm, y_hbm, out_hbm,
                sidx, didx, b0, b1, b2, b3, acc,
                g0, g1, g2, g3, s0, s1, s2, s3):
    cid = lax.axis_index("c")
    sid = lax.axis_index("s")
    wid = cid * NS + sid
    bufs = (b0, b1, b2, b3)
    gsem = (g0, g1, g2, g3)
    ssem = (s0, s1, s2, s3)
    for z in range(ROWS_ACC // ZR):
        pltpu.sync_copy(zeros_hbm, acc.at[pl.ds(sid * ROWS_ACC + z * ZR, ZR)])
    plsc.subcore_barrier()

    @pl.loop(0, NSB2)
    def _(sb):
        pltpu.sync_copy(src_hbm.at[wid, pl.ds(sb * SB, SB)], sidx)
        pltpu.sync_copy(dst_hbm.at[wid, pl.ds(sb * SB, SB)], didx)
        for b in range(NBUF):
            pltpu.async_copy(y_hbm.at[sidx.at[b]], bufs[b], gsem[b])
        for k in range(SB):
            b = k % NBUF
            if k > 0:
                bp = (k - 1) % NBUF
                pltpu.make_async_copy(
                    bufs[bp], acc.at[didx.at[k - 1]], ssem[bp]).wait()
                kn = k + NBUF - 1
                if kn < SB:
                    pltpu.async_copy(y_hbm.at[sidx.at[kn]], bufs[bp],
                                     gsem[bp])
            pltpu.make_async_copy(y_hbm.at[sidx.at[k]], bufs[b],
                                  gsem[b]).wait()
            pltpu.async_copy(bufs[b], acc.at[didx.at[k]], ssem[b], add=True)
        pltpu.make_async_copy(
            bufs[(SB - 1) % NBUF], acc.at[didx.at[SB - 1]],
            ssem[(SB - 1) % NBUF]).wait()

    plsc.subcore_barrier()
    pltpu.sync_copy(acc.at[pl.ds(sid * ROWS_ACC, ROWS_ACC)],
                    out_hbm.at[cid, pl.ds(sid * ROWS_ACC, ROWS_ACC)])


@functools.partial(
    pl.kernel,
    out_type=(jax.ShapeDtypeStruct((E2,), jnp.float32),
              jax.ShapeDtypeStruct((E2,), jnp.float32)),
    mesh=_mesh,
    scratch_types=[
        pltpu.VMEM((SB, CH), jnp.int32),
        pltpu.VMEM((SB, CH), jnp.int32),
        pltpu.VMEM((SB, CH), jnp.int32),
        pltpu.VMEM((CH, D), jnp.float32),
        pltpu.VMEM((CH, D), jnp.float32),
        pltpu.VMEM((CH, D), jnp.float32),
        pltpu.VMEM((CH, D), jnp.float32),
        pltpu.VMEM((CH, D), jnp.float32),
        pltpu.VMEM((CH, D), jnp.float32),
        pltpu.VMEM((D,), jnp.float32),
        pltpu.VMEM((16,), jnp.float32),
        pltpu.VMEM((EPT2,), jnp.float32),
        pltpu.VMEM((EPT2,), jnp.float32),
        pltpu.SemaphoreType.DMA,
        pltpu.SemaphoreType.DMA,
        pltpu.SemaphoreType.DMA,
        pltpu.SemaphoreType.DMA,
        pltpu.SemaphoreType.DMA,
        pltpu.SemaphoreType.DMA,
    ],
    compiler_params=_cp,
)
def _decode_kernel(s_hbm, d_hbm, src_hbm, dst_hbm, neg_hbm, wf_hbm, bf_hbm,
                   pos_hbm, negout_hbm,
                   sidx, didx, nidx, sbuf0, dpbuf0, dnbuf0, sbuf1, dpbuf1,
                   dnbuf1, wf_v, bf_v, pos_v, neg_v,
                   gs0, gp0, gn0, gs1, gp1, gn1):
    cid = lax.axis_index("c")
    sid = lax.axis_index("s")
    wid = cid * NS + sid
    pltpu.sync_copy(wf_hbm, wf_v)
    pltpu.sync_copy(bf_hbm, bf_v)
    lane = lax.iota(jnp.int32, 16)
    bfv = bf_v[...]
    bufs = ((sbuf0, dpbuf0, dnbuf0, gs0, gp0, gn0),
            (sbuf1, dpbuf1, dnbuf1, gs1, gp1, gn1))

    def start(k, b):
        sb_, dp_, dn_, s0, s1, s2 = bufs[b]
        pltpu.async_copy(s_hbm.at[sidx.at[k]], sb_, s0)
        pltpu.async_copy(d_hbm.at[didx.at[k]], dp_, s1)
        pltpu.async_copy(d_hbm.at[nidx.at[k]], dn_, s2)

    def wait(k, b):
        sb_, dp_, dn_, s0, s1, s2 = bufs[b]
        pltpu.make_async_copy(s_hbm.at[sidx.at[k]], sb_, s0).wait()
        pltpu.make_async_copy(d_hbm.at[didx.at[k]], dp_, s1).wait()
        pltpu.make_async_copy(d_hbm.at[nidx.at[k]], dn_, s2).wait()

    wfc = [wf_v[pl.ds(fb * 16, 16)] for fb in range(D // 16)]
    lastlane = lane == 15

    def compute(k_glob, b):
        sb_, dp_, dn_ = bufs[b][:3]

        @pl.loop(0, CH, step=2)
        def _(g):
            base = k_glob * CH + g
            for e in range(2):
                r = g + e
                accp = jnp.zeros((16,), jnp.float32)
                accn = jnp.zeros((16,), jnp.float32)
                for fb in range(D // 16):
                    cs = pl.ds(fb * 16, 16)
                    sc = sb_[r, cs]
                    dp = dp_[r, cs]
                    dn = dn_[r, cs]
                    accp = accp + jnp.maximum(sc + dp, 0.0) * wfc[fb]
                    accn = accn + jnp.maximum(sc + dn, 0.0) * wfc[fb]
                tgt = jnp.full((16,), base + e, jnp.int32)
                plsc.store_scatter(pos_v, [tgt], plsc.cumsum(accp),
                                   mask=lastlane)
                plsc.store_scatter(neg_v, [tgt], plsc.cumsum(accn),
                                   mask=lastlane)

    @pl.loop(0, NSB)
    def _(sb):
        pltpu.sync_copy(src_hbm.at[wid, pl.ds(sb * SB, SB)], sidx)
        pltpu.sync_copy(dst_hbm.at[wid, pl.ds(sb * SB, SB)], didx)
        pltpu.sync_copy(neg_hbm.at[wid, pl.ds(sb * SB, SB)], nidx)
        start(0, 0)
        for k in range(0, SB, 2):
            wait(k, 0)
            start(k + 1, 1)
            compute(sb * SB + k, 0)
            wait(k + 1, 1)
            if k + 2 < SB:
                start(k + 2, 0)
            compute(sb * SB + k + 1, 1)

    @pl.loop(0, EPT2, step=16)
    def _(i):
        sl = pl.ds(i, 16)
        pos_v[sl] = 1.0 / (1.0 + jnp.exp(-(pos_v[sl] + bfv)))
        neg_v[sl] = 1.0 / (1.0 + jnp.exp(-(neg_v[sl] + bfv)))

    pltpu.sync_copy(pos_v, pos_hbm.at[pl.ds(wid * EPT2, EPT2)])
    pltpu.sync_copy(neg_v, negout_hbm.at[pl.ds(wid * EPT2, EPT2)])


def _dinv(h_ref):
    deg = lax.dot_general(h_ref[...], jnp.ones((NW, D), jnp.float32),
                          (((0,), (0,)), ((), ())),
                          preferred_element_type=jnp.float32)
    return lax.rsqrt(1.0 + deg)


def _mm_first_body(x_ref, w_ref, dg_ref, o_ref):
    dinv = _dinv(dg_ref)
    o_ref[...] = jnp.dot(x_ref[...], w_ref[...],
                         preferred_element_type=jnp.float32) * dinv


def _layer_body(agg_ref, y_ref, dg_ref, b_ref, g_ref, be_ref, w_ref, o_ref):
    dinv = _dinv(dg_ref)
    a = agg_ref[...]
    t = (a[0] + a[1] + y_ref[...]) * dinv + b_ref[...]
    x2 = jnp.maximum(t * (g_ref[...] * _BN_SCALE) + be_ref[...], 0.0)
    o_ref[...] = jnp.dot(x2, w_ref[...],
                         preferred_element_type=jnp.float32) * dinv


def _final_body(agg_ref, y_ref, dg_ref, b_ref, ws_ref, bs_ref, wd_ref,
                bd_ref, s_ref, d_ref):
    dinv = _dinv(dg_ref)
    a = agg_ref[...]
    z = (a[0] + a[1] + y_ref[...]) * dinv + b_ref[...]
    s_ref[...] = jnp.dot(z, ws_ref[...],
                         preferred_element_type=jnp.float32) + bs_ref[...]
    d_ref[...] = jnp.dot(z, wd_ref[...],
                         preferred_element_type=jnp.float32) + bd_ref[...]


_row_spec = pl.BlockSpec((BT, D), lambda i: (i, 0))
_w_spec = pl.BlockSpec((D, D), lambda i: (0, 0))
_vec_spec = pl.BlockSpec((1, D), lambda i: (0, 0))
_agg_spec = pl.BlockSpec((NC, BT, D), lambda i: (0, i, 0))
_hist_spec = pl.BlockSpec((NW, BT), lambda i: (0, i))
_row_out = jax.ShapeDtypeStruct((NP_, D), jnp.float32)
_GRID = (NP_ // BT,)


def _mm_first(x, w, dg):
    return pl.pallas_call(
        _mm_first_body, grid=_GRID,
        in_specs=[_row_spec, _w_spec, _hist_spec],
        out_specs=_row_spec, out_shape=_row_out)(x, w, dg)


def _layer(agg, y, dg, b, g, be, w):
    return pl.pallas_call(
        _layer_body, grid=_GRID,
        in_specs=[_agg_spec, _row_spec, _hist_spec, _vec_spec, _vec_spec,
                  _vec_spec, _w_spec],
        out_specs=_row_spec, out_shape=_row_out)(agg, y, dg, b, g, be, w)


def _final(agg, y, dg, b, ws, bs, wd, bd):
    return pl.pallas_call(
        _final_body, grid=_GRID,
        in_specs=[_agg_spec, _row_spec, _hist_spec, _vec_spec, _w_spec,
                  _vec_spec, _w_spec, _vec_spec],
        out_specs=[_row_spec, _row_spec],
        out_shape=[_row_out, _row_out])(agg, y, dg, b, ws, bs, wd, bd)


def _pad_edges(a, fill):
    a = a.reshape(NW, EPT)
    pad = jnp.full((NW, EPT2 - EPT), fill, jnp.int32)
    return jnp.concatenate([a, pad], axis=1).reshape(NW, NCH, CH)


def kernel(node_feat, src, dst, neg, W1, b1, W2, b2, W3, b3, g1, be1,
           g2, be2, Ws, bs, Wd, bd, Wf, bf):
    f32 = jnp.float32
    src = src.astype(jnp.int32)
    dst = dst.astype(jnp.int32)
    neg = neg.astype(jnp.int32)
    x0 = jnp.concatenate(
        [node_feat.astype(f32), jnp.zeros((NP_ - N, D), f32)], axis=0)
    zeros_blk = jnp.zeros((ZR, D), f32)
    src_p = _pad_edges(src, 0)
    dst_p = _pad_edges(dst, PAD_DST)
    neg_p = _pad_edges(neg, 0)
    src_a = src_p.reshape(NW, NCH2, CH2)
    dst_a = dst_p.reshape(NW, NCH2, CH2)
    b1_ = b1.reshape(1, D)
    b2_ = b2.reshape(1, D)
    b3_ = b3.reshape(1, D)
    g1_ = g1.reshape(1, D)
    g2_ = g2.reshape(1, D)
    be1_ = be1.reshape(1, D)
    be2_ = be2.reshape(1, D)
    bs_ = bs.reshape(1, D)
    bd_ = bd.reshape(1, D)
    wf = Wf.reshape(D)
    bfv = jnp.broadcast_to(bf.reshape(1), (16,))

    dg = _deg_kernel(dst_p)
    y1 = _mm_first(x0, W1, dg)
    agg1 = _agg_kernel(zeros_blk, src_a, dst_a, y1)
    y2 = _layer(agg1, y1, dg, b1_, g1_, be1_, W2)
    agg2 = _agg_kernel(zeros_blk, src_a, dst_a, y2)
    y3 = _layer(agg2, y2, dg, b2_, g2_, be2_, W3)
    agg3 = _agg_kernel(zeros_blk, src_a, dst_a, y3)
    s, d = _final(agg3, y3, dg, b3_, Ws, bs_, Wd, bd_)
    pos, negout = _decode_kernel(s, d, src_p, dst_p, neg_p, wf, bfv)
    pos = pos.reshape(NW, EPT2)[:, :EPT].reshape(E, 1)
    negout = negout.reshape(NW, EPT2)[:, :EPT].reshape(E, 1)
    return pos, negout
